# baseline (speedup 1.0000x reference)
import functools

import jax
import jax.numpy as jnp
from jax import lax
from jax.experimental import pallas as pl
from jax.experimental.pallas import tpu as pltpu
from jax.experimental.pallas import tpu_sc as plsc

NC = 2
NS = 16
NW = NC * NS


def _tc_small(h, idx_r, idx_c, W_delta_w, W_entity_w, B, H, M):
    def body(h_ref, ir_ref, ic_ref, wd_ref, we_ref,
             v_ref, g_ref, sidx_ref):
        hh = h_ref[...]
        v_ref[...] = lax.dot_general(hh, wd_ref[...], (((1,), (1,)), ((), ())),
                                     preferred_element_type=jnp.float32)
        g_ref[...] = lax.dot_general(hh, we_ref[...], (((1,), (1,)), ((), ())),
                                     preferred_element_type=jnp.float32)
        ir = ir_ref[...]
        ic = ic_ref[...]
        eq = ir == ic
        bi = lax.broadcasted_iota(jnp.int32, (B, B), 1)
        last = jnp.max(jnp.where(eq, bi, -1), axis=1, keepdims=True)
        rowi = lax.broadcasted_iota(jnp.int32, (B, 1), 0)
        win = last == rowi
        sidx_ref[...] = jnp.where(win, ir, M)

    return pl.pallas_call(
        body,
        out_shape=[
            jax.ShapeDtypeStruct((B, H), jnp.float32),
            jax.ShapeDtypeStruct((B, H), jnp.float32),
            jax.ShapeDtypeStruct((B, 1), jnp.int32),
        ],
    )(h, idx_r, idx_c, W_delta_w, W_entity_w)


def _sc_update_scatter(entities, h, v, bias16, dist_flat, tvals, idx, sidx,
                       B, H, M):
    bpt = B // NW
    HC = H // 16
    mesh = plsc.VectorSubcoreMesh(core_axis_name="c", subcore_axis_name="s")

    @functools.partial(
        pl.kernel,
        out_type=[
            jax.ShapeDtypeStruct((M + 8, H), jnp.float32),
            jax.ShapeDtypeStruct((M,), jnp.float32),
            jax.ShapeDtypeStruct((M,), jnp.float32),
        ],
        mesh=mesh,
        scratch_types=[
            pltpu.VMEM((bpt,), jnp.int32),
            pltpu.VMEM((bpt,), jnp.int32),
            pltpu.VMEM((bpt, H), jnp.float32),
            pltpu.VMEM((bpt, H), jnp.float32),
            pltpu.VMEM((bpt, H), jnp.float32),
            pltpu.VMEM((16,), jnp.float32),
            pltpu.VMEM((M + 16,), jnp.float32),
            pltpu.VMEM((M + 16,), jnp.float32),
            pltpu.VMEM((B,), jnp.int32),
            pltpu.VMEM((B,), jnp.float32),
            pltpu.SemaphoreType.DMA,
            pltpu.SemaphoreType.DMA,
            pltpu.SemaphoreType.DMA,
            pltpu.SemaphoreType.DMA,
            pltpu.SemaphoreType.DMA,
        ],
        compiler_params=pltpu.CompilerParams(needs_layout_passes=False),
    )
    def k(ent_hbm, h_hbm, v_hbm, bias_hbm, dist_hbm, tv_hbm, idx_hbm,
          sidx_hbm, patch_hbm, mask_hbm, dout_hbm,
          idx_v, sidx_v, h_v, v_v, e_v, bias_v, dcol_v, msk_v, asidx_v, tv_v,
          s1, s2, s3, s4, gsem):
        cid = lax.axis_index("c")
        sid = lax.axis_index("s")
        wid = cid * NS + sid
        b0 = wid * bpt
        cp_i = pltpu.async_copy(idx_hbm.at[pl.ds(b0, bpt)], idx_v, s1)
        cp_s = pltpu.async_copy(sidx_hbm.at[pl.ds(b0, bpt)], sidx_v, s2)
        cp_h = pltpu.async_copy(h_hbm.at[pl.ds(b0, bpt)], h_v, s3)
        cp_v = pltpu.async_copy(v_hbm.at[pl.ds(b0, bpt)], v_v, s4)
        pltpu.sync_copy(bias_hbm, bias_v.at[pl.ds(0, 1)])
        cp_i.wait()
        pltpu.async_copy(ent_hbm.at[idx_v], e_v, gsem).wait()
        cp_h.wait()
        cp_v.wait()
        zi = lax.iota(jnp.int32, 16) * 0
        bias = plsc.load_gather(bias_v, [zi])
        ones = jnp.ones((16,), jnp.float32)

        @pl.loop(0, bpt)
        def _row(r):
            acc = jnp.zeros((16,), jnp.float32)
            for kk in range(HC):
                acc = acc + (e_v[r, pl.ds(kk * 16, 16)] *
                             v_v[r, pl.ds(kk * 16, 16)])
            dl = jnp.sum(acc) + bias
            d = 1.0 / (1.0 + jnp.exp(-dl))
            nacc = jnp.zeros((16,), jnp.float32)
            for kk in range(HC):
                u = (d * e_v[r, pl.ds(kk * 16, 16)] +
                     (1.0 - d) * h_v[r, pl.ds(kk * 16, 16)])
                e_v[r, pl.ds(kk * 16, 16)] = u
                nacc = nacc + u * u
            nv = jnp.sum(nacc) * ones
            yi = plsc.bitcast(
                jnp.int32(0x5F3759DF) -
                lax.shift_right_logical(plsc.bitcast(nv, jnp.int32), 1),
                jnp.float32)
            for _ in range(4):
                yi = yi * (1.5 - 0.5 * nv * yi * yi)
            for kk in range(HC):
                e_v[r, pl.ds(kk * 16, 16)] = e_v[r, pl.ds(kk * 16, 16)] * yi

        cp_s.wait()
        pltpu.async_copy(e_v, patch_hbm.at[sidx_v], gsem).wait()

        @pl.when(jnp.logical_and(cid == 0, sid == 0))
        def _mask():
            zv = jnp.zeros((16,), jnp.float32)
            ov = jnp.ones((16,), jnp.float32)

            @pl.loop(0, (M + 16) // 16)
            def _z(i):
                msk_v[pl.ds(i * 16, 16)] = zv

            pltpu.sync_copy(sidx_hbm, asidx_v)
            for j in range(B // 16):
                vi = asidx_v[pl.ds(j * 16, 16)]
                plsc.store_scatter(msk_v, [vi], ov)
            pltpu.sync_copy(msk_v.at[pl.ds(0, M)], mask_hbm)

        @pl.when(jnp.logical_and(cid == 1, sid == 0))
        def _dist():
            pltpu.sync_copy(dist_hbm, dcol_v.at[pl.ds(0, M)])
            pltpu.sync_copy(sidx_hbm, asidx_v)
            pltpu.sync_copy(tv_hbm, tv_v)
            for j in range(B // 16):
                vi = asidx_v[pl.ds(j * 16, 16)]
                vv = tv_v[pl.ds(j * 16, 16)]
                plsc.store_scatter(dcol_v, [vi], vv)
            pltpu.sync_copy(dcol_v.at[pl.ds(0, M)], dout_hbm)

    return k(entities, h, v, bias16, dist_flat, tvals, idx, sidx)


def _tc_score(g, entities, patch, mask2d, drow2d, t, w00, wdb2, web, B, H, M):
    T = 2048

    def body(t_ref, w00_ref, wdb2_ref, web_ref, g_ref, e_ref, p_ref, m_ref,
             d_ref, out_ref):
        sel = jnp.where(m_ref[...] > 0, p_ref[...], e_ref[...])
        blk = lax.dot_general(g_ref[...], sel,
                              (((1,), (1,)), ((), ())),
                              preferred_element_type=jnp.float32)
        mean_t = jnp.sum(t_ref[...], keepdims=True) / B
        w00v = w00_ref[...]
        c0 = wdb2_ref[...] + web_ref[...] - mean_t * w00v
        out_ref[...] = blk + (d_ref[...] * w00v + c0)

    return pl.pallas_call(
        body,
        grid=(M // T,),
        in_specs=[
            pl.BlockSpec((B, 1), lambda i: (0, 0)),
            pl.BlockSpec((1, 1), lambda i: (0, 0)),
            pl.BlockSpec((1, 1), lambda i: (0, 0)),
            pl.BlockSpec((1, 1), lambda i: (0, 0)),
            pl.BlockSpec((B, H), lambda i: (0, 0)),
            pl.BlockSpec((T, H), lambda i: (i, 0)),
            pl.BlockSpec((T, H), lambda i: (i, 0)),
            pl.BlockSpec((T, 1), lambda i: (i, 0)),
            pl.BlockSpec((1, T), lambda i: (0, i)),
        ],
        out_specs=pl.BlockSpec((B, T), lambda i: (0, i)),
        out_shape=jax.ShapeDtypeStruct((B, M), jnp.float32),
        compiler_params=pltpu.CompilerParams(
            dimension_semantics=("parallel",)),
    )(t, w00, wdb2, web, g, entities, patch, mask2d, drow2d)


def kernel(entities, dist_features, h, t, W_delta_w, W_delta_b, W_entity_w,
           W_entity_b, w_dist_w, w_dist_b, idx):
    M, H = entities.shape
    B = h.shape[0]
    idx32 = idx.astype(jnp.int32)
    idx_r = idx32.reshape(B, 1)
    idx_c = idx32.reshape(1, B)

    v, g, sidx = _tc_small(h, idx_r, idx_c, W_delta_w, W_entity_w, B, H, M)
    bias16 = W_delta_b.reshape(1)
    patch, mask, dout = _sc_update_scatter(
        entities, h, v, bias16, dist_features.reshape(M), t.reshape(B),
        idx32, sidx.reshape(B), B, H, M)
    pred = _tc_score(g, entities, patch, mask.reshape(M, 1),
                     dout.reshape(1, M), t,
                     w_dist_w.reshape(1, 1), w_dist_b.reshape(1, 1),
                     W_entity_b.reshape(1, 1), B, H, M)
    return pred


# speedup vs baseline: 1.0446x; 1.0446x over previous
import functools

import jax
import jax.numpy as jnp
from jax import lax
from jax.experimental import pallas as pl
from jax.experimental.pallas import tpu as pltpu
from jax.experimental.pallas import tpu_sc as plsc

NC = 2
NS = 16
NW = NC * NS


def _tc_small(h, idx_r, idx_c, W_delta_w, W_entity_w, B, H, M):
    def body(h_ref, ir_ref, ic_ref, wd_ref, we_ref,
             v_ref, g_ref, sidx_ref):
        hh = h_ref[...]
        v_ref[...] = lax.dot_general(hh, wd_ref[...], (((1,), (1,)), ((), ())),
                                     preferred_element_type=jnp.float32)
        g_ref[...] = lax.dot_general(hh, we_ref[...], (((1,), (1,)), ((), ())),
                                     preferred_element_type=jnp.float32)
        ir = ir_ref[...]
        ic = ic_ref[...]
        eq = ir == ic
        bi = lax.broadcasted_iota(jnp.int32, (B, B), 1)
        last = jnp.max(jnp.where(eq, bi, -1), axis=1, keepdims=True)
        rowi = lax.broadcasted_iota(jnp.int32, (B, 1), 0)
        win = last == rowi
        sidx_ref[...] = jnp.where(win, ir, M)

    return pl.pallas_call(
        body,
        out_shape=[
            jax.ShapeDtypeStruct((B, H), jnp.float32),
            jax.ShapeDtypeStruct((B, H), jnp.float32),
            jax.ShapeDtypeStruct((B, 1), jnp.int32),
        ],
    )(h, idx_r, idx_c, W_delta_w, W_entity_w)


def _sc_update_scatter(entities, h, v, bias16, dist_flat, tvals, idx, sidx,
                       B, H, M):
    bpt = B // NW
    HC = H // 16
    mesh = plsc.VectorSubcoreMesh(core_axis_name="c", subcore_axis_name="s")

    @functools.partial(
        pl.kernel,
        out_type=[
            jax.ShapeDtypeStruct((M + 8, H), jnp.float32),
            jax.ShapeDtypeStruct((M,), jnp.float32),
            jax.ShapeDtypeStruct((M,), jnp.float32),
        ],
        mesh=mesh,
        scratch_types=[
            pltpu.VMEM((bpt,), jnp.int32),
            pltpu.VMEM((bpt,), jnp.int32),
            pltpu.VMEM((bpt, H), jnp.float32),
            pltpu.VMEM((bpt, H), jnp.float32),
            pltpu.VMEM((bpt, H), jnp.float32),
            pltpu.VMEM((16,), jnp.float32),
            pltpu.VMEM((M + 16,), jnp.float32),
            pltpu.VMEM((M + 16,), jnp.float32),
            pltpu.VMEM((B,), jnp.int32),
            pltpu.VMEM((B,), jnp.float32),
            pltpu.SemaphoreType.DMA,
            pltpu.SemaphoreType.DMA,
            pltpu.SemaphoreType.DMA,
            pltpu.SemaphoreType.DMA,
            pltpu.SemaphoreType.DMA,
        ],
        compiler_params=pltpu.CompilerParams(needs_layout_passes=False),
    )
    def k(ent_hbm, h_hbm, v_hbm, bias_hbm, dist_hbm, tv_hbm, idx_hbm,
          sidx_hbm, patch_hbm, mask_hbm, dout_hbm,
          idx_v, sidx_v, h_v, v_v, e_v, bias_v, dcol_v, msk_v, asidx_v, tv_v,
          s1, s2, s3, s4, gsem):
        cid = lax.axis_index("c")
        sid = lax.axis_index("s")
        wid = cid * NS + sid
        b0 = wid * bpt
        cp_i = pltpu.async_copy(idx_hbm.at[pl.ds(b0, bpt)], idx_v, s1)
        cp_s = pltpu.async_copy(sidx_hbm.at[pl.ds(b0, bpt)], sidx_v, s2)
        cp_h = pltpu.async_copy(h_hbm.at[pl.ds(b0, bpt)], h_v, s3)
        cp_v = pltpu.async_copy(v_hbm.at[pl.ds(b0, bpt)], v_v, s4)
        pltpu.sync_copy(bias_hbm, bias_v.at[pl.ds(0, 1)])
        cp_i.wait()
        pltpu.async_copy(ent_hbm.at[idx_v], e_v, gsem).wait()
        cp_h.wait()
        cp_v.wait()
        zi = lax.iota(jnp.int32, 16) * 0
        bias = plsc.load_gather(bias_v, [zi])

        @pl.when(jnp.logical_and(cid == 0, sid == 0))
        def _mask_zero():
            zv = jnp.zeros((16,), jnp.float32)

            @pl.loop(0, M // 64)
            def _z(i):
                for q in range(4):
                    msk_v[pl.ds(i * 64 + q * 16, 16)] = zv

            msk_v[pl.ds(M, 16)] = zv
        ones = jnp.ones((16,), jnp.float32)

        @pl.loop(0, bpt)
        def _row(r):
            acc = jnp.zeros((16,), jnp.float32)
            for kk in range(HC):
                acc = acc + (e_v[r, pl.ds(kk * 16, 16)] *
                             v_v[r, pl.ds(kk * 16, 16)])
            dl = jnp.sum(acc) + bias
            d = 1.0 / (1.0 + jnp.exp(-dl))
            nacc = jnp.zeros((16,), jnp.float32)
            for kk in range(HC):
                u = (d * e_v[r, pl.ds(kk * 16, 16)] +
                     (1.0 - d) * h_v[r, pl.ds(kk * 16, 16)])
                e_v[r, pl.ds(kk * 16, 16)] = u
                nacc = nacc + u * u
            nv = jnp.sum(nacc) * ones
            yi = plsc.bitcast(
                jnp.int32(0x5F3759DF) -
                lax.shift_right_logical(plsc.bitcast(nv, jnp.int32), 1),
                jnp.float32)
            for _ in range(4):
                yi = yi * (1.5 - 0.5 * nv * yi * yi)
            for kk in range(HC):
                e_v[r, pl.ds(kk * 16, 16)] = e_v[r, pl.ds(kk * 16, 16)] * yi

        cp_s.wait()
        pltpu.async_copy(e_v, patch_hbm.at[sidx_v], gsem).wait()

        @pl.when(jnp.logical_and(cid == 0, sid == 0))
        def _mask():
            ov = jnp.ones((16,), jnp.float32)
            pltpu.sync_copy(sidx_hbm, asidx_v)
            for j in range(B // 16):
                vi = asidx_v[pl.ds(j * 16, 16)]
                plsc.store_scatter(msk_v, [vi], ov)
            pltpu.sync_copy(msk_v.at[pl.ds(0, M)], mask_hbm)

        @pl.when(jnp.logical_and(cid == 1, sid == 0))
        def _dist():
            pltpu.sync_copy(dist_hbm, dcol_v.at[pl.ds(0, M)])
            pltpu.sync_copy(sidx_hbm, asidx_v)
            pltpu.sync_copy(tv_hbm, tv_v)
            for j in range(B // 16):
                vi = asidx_v[pl.ds(j * 16, 16)]
                vv = tv_v[pl.ds(j * 16, 16)]
                plsc.store_scatter(dcol_v, [vi], vv)
            pltpu.sync_copy(dcol_v.at[pl.ds(0, M)], dout_hbm)

    return k(entities, h, v, bias16, dist_flat, tvals, idx, sidx)


def _tc_score(g, entities, patch, mask2d, drow2d, t, w00, wdb2, web, B, H, M):
    T = 2048

    def body(t_ref, w00_ref, wdb2_ref, web_ref, g_ref, e_ref, p_ref, m_ref,
             d_ref, out_ref):
        sel = jnp.where(m_ref[...] > 0, p_ref[...], e_ref[...])
        blk = lax.dot_general(g_ref[...], sel,
                              (((1,), (1,)), ((), ())),
                              preferred_element_type=jnp.float32)
        mean_t = jnp.sum(t_ref[...], keepdims=True) / B
        w00v = w00_ref[...]
        c0 = wdb2_ref[...] + web_ref[...] - mean_t * w00v
        out_ref[...] = blk + (d_ref[...] * w00v + c0)

    return pl.pallas_call(
        body,
        grid=(M // T,),
        in_specs=[
            pl.BlockSpec((B, 1), lambda i: (0, 0)),
            pl.BlockSpec((1, 1), lambda i: (0, 0)),
            pl.BlockSpec((1, 1), lambda i: (0, 0)),
            pl.BlockSpec((1, 1), lambda i: (0, 0)),
            pl.BlockSpec((B, H), lambda i: (0, 0)),
            pl.BlockSpec((T, H), lambda i: (i, 0)),
            pl.BlockSpec((T, H), lambda i: (i, 0)),
            pl.BlockSpec((T, 1), lambda i: (i, 0)),
            pl.BlockSpec((1, T), lambda i: (0, i)),
        ],
        out_specs=pl.BlockSpec((B, T), lambda i: (0, i)),
        out_shape=jax.ShapeDtypeStruct((B, M), jnp.float32),
        compiler_params=pltpu.CompilerParams(
            dimension_semantics=("parallel",)),
    )(t, w00, wdb2, web, g, entities, patch, mask2d, drow2d)


def kernel(entities, dist_features, h, t, W_delta_w, W_delta_b, W_entity_w,
           W_entity_b, w_dist_w, w_dist_b, idx):
    M, H = entities.shape
    B = h.shape[0]
    idx32 = idx.astype(jnp.int32)
    idx_r = idx32.reshape(B, 1)
    idx_c = idx32.reshape(1, B)

    v, g, sidx = _tc_small(h, idx_r, idx_c, W_delta_w, W_entity_w, B, H, M)
    bias16 = W_delta_b.reshape(1)
    patch, mask, dout = _sc_update_scatter(
        entities, h, v, bias16, dist_features.reshape(M), t.reshape(B),
        idx32, sidx.reshape(B), B, H, M)
    pred = _tc_score(g, entities, patch, mask.reshape(M, 1),
                     dout.reshape(1, M), t,
                     w_dist_w.reshape(1, 1), w_dist_b.reshape(1, 1),
                     W_entity_b.reshape(1, 1), B, H, M)
    return pred
